# baseline (speedup 1.0000x reference)
import jax
import jax.numpy as jnp
from jax import lax
from jax.experimental import pallas as pl
from jax.experimental.pallas import tpu as pltpu
from jax.experimental.pallas import tpu_sc as plsc

_K = 32
_B = 16384
_NC = 2
_NS = 16
_NW = _NC * _NS
_BPW = _B // _NW
_L = 16
_C = 128
_NCH = _BPW // _C


def _body(users_hbm, items_hbm, neg_hbm, p_hbm, q_hbm, pos_out, neg_out,
          idx_u, idx_i, idx_n, ru, ri, rn, pos_v, neg_v, sem):
    wid = lax.axis_index("s") * _NC + lax.axis_index("c")
    base = wid * _BPW

    pltpu.sync_copy(users_hbm.at[pl.ds(wid * _NCH, _NCH)], idx_u)
    pltpu.sync_copy(items_hbm.at[pl.ds(wid * _NCH, _NCH)], idx_i)
    pltpu.sync_copy(neg_hbm.at[pl.ds(wid * _NCH, _NCH)], idx_n)

    copies = []
    for j in range(_NCH):
        d = pl.ds(j * _C, _C)
        copies.append(pltpu.async_copy(p_hbm.at[idx_u.at[j]], ru.at[d], sem))
        copies.append(pltpu.async_copy(q_hbm.at[idx_i.at[j]], ri.at[d], sem))
        copies.append(pltpu.async_copy(q_hbm.at[idx_n.at[j]], rn.at[d], sem))
    for c in copies:
        c.wait()

    row16 = lax.iota(jnp.int32, _L)

    def group(g, carry):
        rows = g * _L + row16
        acc_p = jnp.zeros((_L,), jnp.float32)
        acc_n = jnp.zeros((_L,), jnp.float32)
        for k in range(_K):
            kk = jnp.full((_L,), k, jnp.int32)
            u = plsc.load_gather(ru, [rows, kk])
            qi = plsc.load_gather(ri, [rows, kk])
            qn = plsc.load_gather(rn, [rows, kk])
            acc_p = acc_p + u * qi
            acc_n = acc_n + u * qn
        s = pl.ds(g * _L, _L)
        pos_v[s] = acc_p
        neg_v[s] = acc_n
        return carry

    lax.fori_loop(0, _BPW // _L, group, 0)

    pltpu.sync_copy(pos_v, pos_out.at[pl.ds(base, _BPW)])
    pltpu.sync_copy(neg_v, neg_out.at[pl.ds(base, _BPW)])


@jax.jit
def _run(users, items, neg_items, p, q):
    mesh = plsc.VectorSubcoreMesh(core_axis_name="c", subcore_axis_name="s")
    f = pl.kernel(
        _body,
        mesh=mesh,
        out_type=(
            jax.ShapeDtypeStruct((_B,), jnp.float32),
            jax.ShapeDtypeStruct((_B,), jnp.float32),
        ),
        scratch_types=[
            pltpu.VMEM((_NCH, _C), jnp.int32),
            pltpu.VMEM((_NCH, _C), jnp.int32),
            pltpu.VMEM((_NCH, _C), jnp.int32),
            pltpu.VMEM((_BPW, _K), jnp.float32),
            pltpu.VMEM((_BPW, _K), jnp.float32),
            pltpu.VMEM((_BPW, _K), jnp.float32),
            pltpu.VMEM((_BPW,), jnp.float32),
            pltpu.VMEM((_BPW,), jnp.float32),
            pltpu.SemaphoreType.DMA,
        ],
        compiler_params=pltpu.CompilerParams(
            needs_layout_passes=False, use_tc_tiling_on_sc=False
        ),
    )
    u2 = users.reshape(_B // _C, _C)
    i2 = items.reshape(_B // _C, _C)
    n2 = neg_items.reshape(_B // _C, _C)
    return f(u2, i2, n2, p, q)


def kernel(users, items, neg_items, P, Q):
    users = users.astype(jnp.int32)
    items = items.astype(jnp.int32)
    neg_items = neg_items.astype(jnp.int32)
    return _run(users, items, neg_items, P, Q)
